# Initial kernel scaffold; baseline (speedup 1.0000x reference)
import jax
import jax.numpy as jnp
from jax.experimental import pallas as pl


def kernel(x, edge_index, W1, b1, W2, b2):
    raise NotImplementedError("write your pallas kernel here")


# speedup vs baseline: 9.2991x; 9.2991x over previous
import functools

import jax
import jax.numpy as jnp
from jax import lax
from jax.experimental import pallas as pl
from jax.experimental.pallas import tpu as pltpu
from jax.experimental.pallas import tpu_sc as plsc

_L = 16
_CHUNK = 128
_NCORES = 2
_NSUB = 16
_NTILES = _NCORES * _NSUB
_NBUF = 2
_CHP = 64
_NHALF = 2
_R = 512


def _rsqrt_newton(d):
    i = lax.bitcast_convert_type(d, jnp.int32)
    y = lax.bitcast_convert_type(jnp.int32(0x5F3759DF) - (i >> 1), jnp.float32)
    for _ in range(3):
        y = y * (1.5 - 0.5 * d * y * y)
    return y


def _deg_scale_kernel(np_, d, ch1):
    npt = np_ // _NTILES
    nps = np_ // _NSUB
    mesh = plsc.VectorSubcoreMesh(core_axis_name="c", subcore_axis_name="s")

    @functools.partial(
        pl.kernel,
        out_type=(
            jax.ShapeDtypeStruct((np_,), jnp.float32),
            jax.ShapeDtypeStruct((np_, d), jnp.float32),
        ),
        mesh=mesh,
        scratch_types=(
            pltpu.VMEM((ch1, _CHUNK), jnp.int32),
            pltpu.VMEM((_CHUNK,), jnp.float32),
            pltpu.VMEM((nps,), jnp.float32),
            pltpu.VMEM((npt,), jnp.float32),
            pltpu.VMEM((npt,), jnp.float32),
            pltpu.VMEM((npt, d), jnp.float32),
            pltpu.VMEM_SHARED((np_,), jnp.float32),
        ),
        compiler_params=pltpu.CompilerParams(needs_layout_passes=False),
    )
    def deg_scale(col_hbm, x_hbm, dinv_hbm, x1_hbm,
                  idx_v, ones_v, z_v, deg_v, dinv_v, xt_v, deg_acc):
        c = lax.axis_index("c")
        s = lax.axis_index("s")

        @pl.loop(0, nps // _L)
        def _zfill(i):
            z_v[pl.ds(i * _L, _L)] = jnp.zeros((_L,), jnp.float32)

        for i in range(_CHUNK // _L):
            ones_v[pl.ds(i * _L, _L)] = jnp.ones((_L,), jnp.float32)

        pltpu.sync_copy(z_v, deg_acc.at[pl.ds(s * nps, nps)])
        pltpu.sync_copy(col_hbm.at[s], idx_v)
        plsc.subcore_barrier()

        @pl.loop(0, ch1)
        def _count(j):
            pltpu.sync_copy(ones_v, deg_acc.at[idx_v.at[j]], add=True)

        plsc.subcore_barrier()

        off = (c * _NSUB + s) * npt
        pltpu.sync_copy(deg_acc.at[pl.ds(off, npt)], deg_v)
        pltpu.sync_copy(x_hbm.at[pl.ds(off, npt)], xt_v)

        @pl.loop(0, npt // _L)
        def _dinv(i):
            dv = deg_v[pl.ds(i * _L, _L)] + 1.0
            dinv_v[pl.ds(i * _L, _L)] = _rsqrt_newton(dv)

        @pl.loop(0, npt)
        def _scale(r):
            idx = jnp.full((_L,), r, jnp.int32)
            sp = plsc.load_gather(dinv_v, [idx])
            for k in range(d // _L):
                xt_v[r, pl.ds(k * _L, _L)] = xt_v[r, pl.ds(k * _L, _L)] * sp

        pltpu.sync_copy(dinv_v, dinv_hbm.at[pl.ds(off, npt)])
        pltpu.sync_copy(xt_v, x1_hbm.at[pl.ds(off, npt)])

    return deg_scale


def _propagate_kernel(np_, d, ch2):
    nps = np_ // _NSUB
    chb = ch2 // _NHALF
    mesh = plsc.VectorSubcoreMesh(core_axis_name="c", subcore_axis_name="s")

    @functools.partial(
        pl.kernel,
        out_type=jax.ShapeDtypeStruct((_NCORES, np_, d), jnp.float32),
        mesh=mesh,
        scratch_types=(
            pltpu.VMEM((chb, _CHP), jnp.int32),
            pltpu.VMEM((chb, _CHP), jnp.int32),
            pltpu.VMEM((_NBUF, _CHP, d), jnp.float32),
            pltpu.VMEM_SHARED((np_, d), jnp.float32),
            *([pltpu.SemaphoreType.DMA] * (2 * _NBUF)),
        ),
        compiler_params=pltpu.CompilerParams(needs_layout_passes=False),
    )
    def propagate(src_hbm, row_hbm, col_hbm, out_hbm,
                  rbuf, cbuf, dbuf, acc, *sems):
        gsem = sems[:_NBUF]
        ssem = sems[_NBUF:]
        c = lax.axis_index("c")
        s = lax.axis_index("s")
        t = c * _NSUB + s

        @pl.loop(0, _CHP)
        def _zfill(i):
            for k in range(d // _L):
                dbuf[0, i, pl.ds(k * _L, _L)] = jnp.zeros((_L,), jnp.float32)

        for q in range(nps // _CHP):
            pltpu.sync_copy(dbuf.at[0], acc.at[pl.ds(s * nps + q * _CHP, _CHP)])
        plsc.subcore_barrier()

        for half in range(_NHALF):
            pltpu.sync_copy(row_hbm.at[t, pl.ds(half * chb, chb)], rbuf)
            pltpu.sync_copy(col_hbm.at[t, pl.ds(half * chb, chb)], cbuf)
            for k in range(_NBUF):
                pltpu.async_copy(src_hbm.at[rbuf.at[k]], dbuf.at[k], gsem[k])

            @pl.loop(0, chb, step=_NBUF)
            def _edges(j):
                for k in range(_NBUF):
                    jj = j + k
                    pltpu.make_async_copy(
                        src_hbm.at[rbuf.at[jj]], dbuf.at[k], gsem[k]).wait()
                    pltpu.async_copy(
                        dbuf.at[k], acc.at[cbuf.at[jj]], ssem[k], add=True)
                for k in range(_NBUF):
                    jj = j + k
                    pltpu.make_async_copy(
                        dbuf.at[k], acc.at[cbuf.at[jj]], ssem[k]).wait()

                    @pl.when(jj + _NBUF < chb)
                    def _next():
                        pltpu.async_copy(
                            src_hbm.at[rbuf.at[jj + _NBUF]], dbuf.at[k],
                            gsem[k])

        plsc.subcore_barrier()
        pltpu.sync_copy(acc.at[pl.ds(s * nps, nps)],
                        out_hbm.at[c, pl.ds(s * nps, nps)])

    return propagate


def _mm_kernel(np_, d, dh):
    grid = (np_ // _R,)

    def body(s1_ref, x_ref, dinv_ref, w1_ref, b1_ref, w2_ref, b2_ref,
             y_ref, t_ref):
        di = dinv_ref[...]
        di2 = di * di
        p = di * (s1_ref[0] + s1_ref[1]) + di2 * x_ref[...]
        h = jnp.dot(p, w1_ref[...], preferred_element_type=jnp.float32)
        h = jnp.maximum(h + b1_ref[...], 0.0)
        g = jnp.dot(h, w2_ref[...], preferred_element_type=jnp.float32)
        y_ref[...] = di * g
        t_ref[...] = di2 * g + b2_ref[...]

    return pl.pallas_call(
        body,
        grid=grid,
        in_specs=[
            pl.BlockSpec((_NCORES, _R, d), lambda i: (0, i, 0)),
            pl.BlockSpec((_R, d), lambda i: (i, 0)),
            pl.BlockSpec((_R, 1), lambda i: (i, 0)),
            pl.BlockSpec((d, dh), lambda i: (0, 0)),
            pl.BlockSpec((1, dh), lambda i: (0, 0)),
            pl.BlockSpec((dh, d), lambda i: (0, 0)),
            pl.BlockSpec((1, d), lambda i: (0, 0)),
        ],
        out_specs=[
            pl.BlockSpec((_R, d), lambda i: (i, 0)),
            pl.BlockSpec((_R, d), lambda i: (i, 0)),
        ],
        out_shape=[
            jax.ShapeDtypeStruct((np_, d), jnp.float32),
            jax.ShapeDtypeStruct((np_, d), jnp.float32),
        ],
    )


def _final_kernel(np_, d):
    grid = (np_ // _R,)

    def body(s2_ref, dinv_ref, t_ref, o_ref):
        o_ref[...] = dinv_ref[...] * (s2_ref[0] + s2_ref[1]) + t_ref[...]

    return pl.pallas_call(
        body,
        grid=grid,
        in_specs=[
            pl.BlockSpec((_NCORES, _R, d), lambda i: (0, i, 0)),
            pl.BlockSpec((_R, 1), lambda i: (i, 0)),
            pl.BlockSpec((_R, d), lambda i: (i, 0)),
        ],
        out_specs=pl.BlockSpec((_R, d), lambda i: (i, 0)),
        out_shape=jax.ShapeDtypeStruct((np_, d), jnp.float32),
    )


def kernel(x, edge_index, W1, b1, W2, b2):
    n, d = x.shape
    dh = W1.shape[1]
    e = edge_index.shape[1]

    tile_n = _NTILES * _L * 2
    np_ = ((n + tile_n - 1) // tile_n) * tile_n
    tile_e = _NTILES * _CHP * _NBUF * _NHALF
    ep = ((e + tile_e - 1) // tile_e) * tile_e
    ch2 = ep // (_NTILES * _CHP)
    ch1 = ep // (_NSUB * _CHUNK)

    pad_node = np_ - 1
    epad = jnp.full((ep - e,), pad_node, jnp.int32)
    rowp = jnp.concatenate([edge_index[0], epad]).reshape(_NTILES, ch2, _CHP)
    colp = jnp.concatenate([edge_index[1], epad])
    colp2 = colp.reshape(_NTILES, ch2, _CHP)
    colp1 = colp.reshape(_NSUB, ch1, _CHUNK)
    xp = jnp.concatenate([x, jnp.zeros((np_ - n, d), x.dtype)])

    dinv, x1 = _deg_scale_kernel(np_, d, ch1)(colp1, xp)
    dinv = dinv.reshape(np_, 1)
    prop = _propagate_kernel(np_, d, ch2)
    s1 = prop(x1, rowp, colp2)
    y, t = _mm_kernel(np_, d, dh)(
        s1, xp, dinv, W1, b1.reshape(1, dh), W2, b2.reshape(1, d))
    s2 = prop(y, rowp, colp2)
    out = _final_kernel(np_, d)(s2, dinv, t)
    return out[:n]
